# baseline (speedup 1.0000x reference)
import jax
import jax.numpy as jnp
from jax import lax
from jax.experimental import pallas as pl
from jax.experimental.pallas import tpu as pltpu
from jax.experimental.pallas import tpu_sc as plsc

N = 10000
E = 320000
D = 128
NC, NS = 2, 16
K = 128
CPW0 = 160
IB = 16
DIB = 16
DCW0 = 112
DCW1 = 48
E_PAD = NS * CPW0 * K
NP = 10240
RPT = NP // NS
HB = NP // K
GB = NP // 1024

_MESH = plsc.VectorSubcoreMesh(
    core_axis_name="c", subcore_axis_name="s", num_cores=NC, num_subcores=NS)


def _worker_layout(c, s):
    rbase = s * CPW0
    nblk = jnp.where(c == 0, CPW0 // IB, 0)
    return rbase, nblk


def _deg_body(src2_hbm, dst2_hbm, zer2_hbm, iden_hbm, out_s, out_d,
              sh_s, sh_d, hs_v, hd_v, sidx, didx, iden_v, tbuf):
    c = lax.axis_index("c")
    s = lax.axis_index("s")
    rbase = jnp.where(c == 0, s * DCW0, NS * DCW0 + s * DCW1)
    nblk = jnp.where(c == 0, DCW0 // DIB, DCW1 // DIB)
    pltpu.sync_copy(zer2_hbm, hs_v)
    pltpu.sync_copy(zer2_hbm, hd_v)
    pltpu.sync_copy(iden_hbm, iden_v)

    @pl.when(s < HB // 8)
    def _():
        pltpu.sync_copy(hs_v.at[pl.ds(s * 8, 8)], sh_s.at[pl.ds(s * 8, 8)])
        pltpu.sync_copy(hd_v.at[pl.ds(s * 8, 8)], sh_d.at[pl.ds(s * 8, 8)])

    ones = jnp.ones((16,), jnp.float32)

    def blk(bi, carry):
        pltpu.sync_copy(src2_hbm.at[pl.ds(rbase + bi * DIB, DIB)], sidx)
        pltpu.sync_copy(dst2_hbm.at[pl.ds(rbase + bi * DIB, DIB)], didx)

        def step(i, carry2):
            for j in range(K // 16):
                si = sidx[i, pl.ds(j * 16, 16)]
                plsc.addupdate_scatter(
                    hs_v,
                    [lax.shift_right_logical(si, 7), lax.bitwise_and(si, 127)],
                    ones)
                di = didx[i, pl.ds(j * 16, 16)]
                plsc.addupdate_scatter(
                    hd_v,
                    [lax.shift_right_logical(di, 7), lax.bitwise_and(di, 127)],
                    ones)
            return carry2

        lax.fori_loop(0, DIB, step, 0)
        return carry

    lax.fori_loop(0, nblk, blk, 0)
    plsc.subcore_barrier()
    pltpu.sync_copy(hs_v, sh_s.at[iden_v], add=True)
    pltpu.sync_copy(hd_v, sh_d.at[iden_v], add=True)
    plsc.subcore_barrier()

    @pl.when(s < HB // 8)
    def _():
        pltpu.sync_copy(sh_s.at[pl.ds(s * 8, 8)], tbuf)
        pltpu.sync_copy(tbuf, out_s.at[pl.ds(c * HB + s * 8, 8)])
        pltpu.sync_copy(sh_d.at[pl.ds(s * 8, 8)], tbuf)
        pltpu.sync_copy(tbuf, out_d.at[pl.ds(c * HB + s * 8, 8)])


_deg_kernel = pl.kernel(
    _deg_body,
    out_type=[jax.ShapeDtypeStruct((NC * HB, 128), jnp.float32),
              jax.ShapeDtypeStruct((NC * HB, 128), jnp.float32)],
    mesh=_MESH,
    scratch_types=[
        pltpu.VMEM_SHARED((HB, 128), jnp.float32),
        pltpu.VMEM_SHARED((HB, 128), jnp.float32),
        pltpu.VMEM((HB, 128), jnp.float32),
        pltpu.VMEM((HB, 128), jnp.float32),
        pltpu.VMEM((DIB, K), jnp.int32),
        pltpu.VMEM((DIB, K), jnp.int32),
        pltpu.VMEM((HB,), jnp.int32),
        pltpu.VMEM((8, 128), jnp.float32),
    ],
    compiler_params=pltpu.CompilerParams(needs_layout_passes=False),
)


def _edge_body(src2_hbm, dst2_hbm, h_hbm, zer2_hbm, out_acc,
               acc, sidx, didx, rows0, rows1, g0, g1):
    c = lax.axis_index("c")
    s = lax.axis_index("s")
    rbase, nblk = _worker_layout(c, s)
    with jax.named_scope("zero_acc"):
        @pl.when(c == 0)
        def _():
            pltpu.sync_copy(zer2_hbm, rows0)
            for j in range(RPT // K):
                pltpu.sync_copy(rows0, acc.at[pl.ds(s * RPT + j * K, K)])

        plsc.subcore_barrier()

    def blk(bi, carry):
        pltpu.sync_copy(src2_hbm.at[pl.ds(rbase + bi * IB, IB)], sidx)
        pltpu.sync_copy(dst2_hbm.at[pl.ds(rbase + bi * IB, IB)], didx)
        pltpu.async_copy(h_hbm.at[sidx.at[0]], rows0, g0)

        def step2(i2, carry2):
            i0 = 2 * i2
            pltpu.async_copy(h_hbm.at[sidx.at[i0 + 1]], rows1, g1)
            pltpu.make_async_copy(h_hbm.at[sidx.at[i0]], rows0, g0).wait()
            pltpu.sync_copy(rows0, acc.at[didx.at[i0]], add=True)

            @pl.when(i0 + 2 < IB)
            def _():
                pltpu.async_copy(h_hbm.at[sidx.at[i0 + 2]], rows0, g0)

            pltpu.make_async_copy(h_hbm.at[sidx.at[i0 + 1]], rows1, g1).wait()
            pltpu.sync_copy(rows1, acc.at[didx.at[i0 + 1]], add=True)
            return carry2

        lax.fori_loop(0, IB // 2, step2, 0)
        return carry

    with jax.named_scope("chunks"):
        lax.fori_loop(0, nblk, blk, 0)
        plsc.subcore_barrier()
    with jax.named_scope("writeback"):
        @pl.when(c == 0)
        def _():
            for j in range(RPT // K):
                pltpu.sync_copy(acc.at[pl.ds(s * RPT + j * K, K)], rows0)
                pltpu.sync_copy(
                    rows0, out_acc.at[pl.ds(s * RPT + j * K, K)])


_edge_kernel = pl.kernel(
    _edge_body,
    out_type=jax.ShapeDtypeStruct((NP, D), jnp.float32),
    mesh=_MESH,
    scratch_types=[
        pltpu.VMEM_SHARED((NP, D), jnp.float32),
        pltpu.VMEM((IB, K), jnp.int32),
        pltpu.VMEM((IB, K), jnp.int32),
        pltpu.VMEM((K, D), jnp.float32),
        pltpu.VMEM((K, D), jnp.float32),
        pltpu.SemaphoreType.DMA,
        pltpu.SemaphoreType.DMA,
    ],
)


def _scale_body(f_ref, h0_ref, h1_ref, o_ref):
    cnt = h0_ref[0, 0, :] + h1_ref[0, 0, :]
    scale = lax.rsqrt(jnp.maximum(cnt, 1.0))
    o_ref[...] = f_ref[...] * scale[:, None]


def _out_body(a_ref, h0_ref, h1_ref, w_ref, b_ref, o_ref):
    cnt = h0_ref[0, 0, :] + h1_ref[0, 0, :]
    inv = lax.rsqrt(jnp.maximum(cnt, 1.0))
    x = a_ref[...] * inv[:, None]
    y = jnp.dot(x, w_ref[...], preferred_element_type=jnp.float32)
    o_ref[...] = jnp.maximum(y + b_ref[0:1, :], 0.0)


def kernel(feature, edge_index, W, b):
    src = edge_index[0]
    dst = edge_index[1]
    pad = jnp.full((E_PAD - E,), NP - 1, dtype=jnp.int32)
    src2 = jnp.concatenate([src, pad]).reshape(E_PAD // K, K)
    dst2 = jnp.concatenate([dst, pad]).reshape(E_PAD // K, K)
    feature_p = jnp.pad(feature, ((0, NP - N), (0, 0)))
    zer_h = jnp.zeros((HB, 128), dtype=jnp.float32)
    zer_r = jnp.zeros((K, D), dtype=jnp.float32)
    iden = jnp.arange(HB, dtype=jnp.int32)
    b2 = jnp.broadcast_to(b, (8, D))

    hist_s, hist_d = _deg_kernel(src2, dst2, zer_h, iden)
    hist_s3 = hist_s.reshape(NC * GB, 1, 1024)
    hist_d3 = hist_d.reshape(NC * GB, 1, 1024)

    h = pl.pallas_call(
        _scale_body,
        grid=(GB,),
        in_specs=[pl.BlockSpec((1024, D), lambda i: (i, 0)),
                  pl.BlockSpec((1, 1, 1024), lambda i: (i, 0, 0)),
                  pl.BlockSpec((1, 1, 1024), lambda i: (i + GB, 0, 0))],
        out_specs=pl.BlockSpec((1024, D), lambda i: (i, 0)),
        out_shape=jax.ShapeDtypeStruct((NP, D), jnp.float32),
    )(feature_p, hist_s3, hist_s3)

    acc = _edge_kernel(src2, dst2, h, zer_r)

    out = pl.pallas_call(
        _out_body,
        grid=(GB,),
        in_specs=[pl.BlockSpec((1024, D), lambda i: (i, 0)),
                  pl.BlockSpec((1, 1, 1024), lambda i: (i, 0, 0)),
                  pl.BlockSpec((1, 1, 1024), lambda i: (i + GB, 0, 0)),
                  pl.BlockSpec((128, D), lambda i: (0, 0)),
                  pl.BlockSpec((8, D), lambda i: (0, 0))],
        out_specs=pl.BlockSpec((1024, D), lambda i: (i, 0)),
        out_shape=jax.ShapeDtypeStruct((NP, D), jnp.float32),
    )(acc, hist_d3, hist_d3, W, b2)

    return out[:N]


# speedup vs baseline: 3.1784x; 3.1784x over previous
import jax
import jax.numpy as jnp
from jax import lax
from jax.experimental import pallas as pl
from jax.experimental.pallas import tpu as pltpu
from jax.experimental.pallas import tpu_sc as plsc

N = 10000
E = 320000
D = 128
NC, NS = 2, 16
K = 128
CPW = 80
IB = 16
DIB = 16
E_PAD = NC * NS * CPW * K
NP = 10240
RPT = NP // NS
HB = NP // K
GB = NP // 1024

_MESH = plsc.VectorSubcoreMesh(
    core_axis_name="c", subcore_axis_name="s", num_cores=NC, num_subcores=NS)


def _worker_layout(c, s):
    rbase = (c * NS + s) * CPW
    nblk = CPW // IB
    return rbase, nblk


def _deg_body(src2_hbm, dst2_hbm, zer2_hbm, iden_hbm, out_s, out_d,
              sh_s, sh_d, hs_v, hd_v, sidx, didx, iden_v, tbuf):
    c = lax.axis_index("c")
    s = lax.axis_index("s")
    rbase, _ = _worker_layout(c, s)
    nblk = CPW // DIB
    pltpu.sync_copy(zer2_hbm, hs_v)
    pltpu.sync_copy(zer2_hbm, hd_v)
    pltpu.sync_copy(iden_hbm, iden_v)

    @pl.when(s < HB // 8)
    def _():
        pltpu.sync_copy(hs_v.at[pl.ds(s * 8, 8)], sh_s.at[pl.ds(s * 8, 8)])
        pltpu.sync_copy(hd_v.at[pl.ds(s * 8, 8)], sh_d.at[pl.ds(s * 8, 8)])

    ones = jnp.ones((16,), jnp.float32)

    def blk(bi, carry):
        pltpu.sync_copy(src2_hbm.at[pl.ds(rbase + bi * DIB, DIB)], sidx)
        pltpu.sync_copy(dst2_hbm.at[pl.ds(rbase + bi * DIB, DIB)], didx)

        def step(i, carry2):
            for j in range(K // 16):
                si = sidx[i, pl.ds(j * 16, 16)]
                plsc.addupdate_scatter(
                    hs_v,
                    [lax.shift_right_logical(si, 7), lax.bitwise_and(si, 127)],
                    ones)
                di = didx[i, pl.ds(j * 16, 16)]
                plsc.addupdate_scatter(
                    hd_v,
                    [lax.shift_right_logical(di, 7), lax.bitwise_and(di, 127)],
                    ones)
            return carry2

        lax.fori_loop(0, DIB, step, 0)
        return carry

    lax.fori_loop(0, nblk, blk, 0)
    plsc.subcore_barrier()
    pltpu.sync_copy(hs_v, sh_s.at[iden_v], add=True)
    pltpu.sync_copy(hd_v, sh_d.at[iden_v], add=True)
    plsc.subcore_barrier()

    @pl.when(s < HB // 8)
    def _():
        pltpu.sync_copy(sh_s.at[pl.ds(s * 8, 8)], tbuf)
        pltpu.sync_copy(tbuf, out_s.at[pl.ds(c * HB + s * 8, 8)])
        pltpu.sync_copy(sh_d.at[pl.ds(s * 8, 8)], tbuf)
        pltpu.sync_copy(tbuf, out_d.at[pl.ds(c * HB + s * 8, 8)])


_deg_kernel = pl.kernel(
    _deg_body,
    out_type=[jax.ShapeDtypeStruct((NC * HB, 128), jnp.float32),
              jax.ShapeDtypeStruct((NC * HB, 128), jnp.float32)],
    mesh=_MESH,
    scratch_types=[
        pltpu.VMEM_SHARED((HB, 128), jnp.float32),
        pltpu.VMEM_SHARED((HB, 128), jnp.float32),
        pltpu.VMEM((HB, 128), jnp.float32),
        pltpu.VMEM((HB, 128), jnp.float32),
        pltpu.VMEM((DIB, K), jnp.int32),
        pltpu.VMEM((DIB, K), jnp.int32),
        pltpu.VMEM((HB,), jnp.int32),
        pltpu.VMEM((8, 128), jnp.float32),
    ],
    compiler_params=pltpu.CompilerParams(needs_layout_passes=False),
)


def _edge_body(src2_hbm, dst2_hbm, h_hbm, zer2_hbm, out_acc,
               acc, sidx, didx, rows0, rows1, g0, g1):
    c = lax.axis_index("c")
    s = lax.axis_index("s")
    rbase, nblk = _worker_layout(c, s)
    with jax.named_scope("zero_acc"):
        pltpu.sync_copy(zer2_hbm, rows0)
        for j in range(RPT // K):
            pltpu.sync_copy(rows0, acc.at[pl.ds(s * RPT + j * K, K)])
        plsc.subcore_barrier()

    def blk(bi, carry):
        pltpu.sync_copy(src2_hbm.at[pl.ds(rbase + bi * IB, IB)], sidx)
        pltpu.sync_copy(dst2_hbm.at[pl.ds(rbase + bi * IB, IB)], didx)
        pltpu.async_copy(h_hbm.at[sidx.at[0]], rows0, g0)

        def step2(i2, carry2):
            i0 = 2 * i2
            pltpu.async_copy(h_hbm.at[sidx.at[i0 + 1]], rows1, g1)
            pltpu.make_async_copy(h_hbm.at[sidx.at[i0]], rows0, g0).wait()
            pltpu.sync_copy(rows0, acc.at[didx.at[i0]], add=True)

            @pl.when(i0 + 2 < IB)
            def _():
                pltpu.async_copy(h_hbm.at[sidx.at[i0 + 2]], rows0, g0)

            pltpu.make_async_copy(h_hbm.at[sidx.at[i0 + 1]], rows1, g1).wait()
            pltpu.sync_copy(rows1, acc.at[didx.at[i0 + 1]], add=True)
            return carry2

        lax.fori_loop(0, IB // 2, step2, 0)
        return carry

    with jax.named_scope("chunks"):
        lax.fori_loop(0, nblk, blk, 0)
        plsc.subcore_barrier()
    with jax.named_scope("writeback"):
        for j in range(RPT // K):
            pltpu.sync_copy(acc.at[pl.ds(s * RPT + j * K, K)], rows0)
            pltpu.sync_copy(
                rows0, out_acc.at[pl.ds(c * NP + s * RPT + j * K, K)])


_edge_kernel = pl.kernel(
    _edge_body,
    out_type=jax.ShapeDtypeStruct((NC * NP, D), jnp.float32),
    mesh=_MESH,
    scratch_types=[
        pltpu.VMEM_SHARED((NP, D), jnp.float32),
        pltpu.VMEM((IB, K), jnp.int32),
        pltpu.VMEM((IB, K), jnp.int32),
        pltpu.VMEM((K, D), jnp.float32),
        pltpu.VMEM((K, D), jnp.float32),
        pltpu.SemaphoreType.DMA,
        pltpu.SemaphoreType.DMA,
    ],
)


def _scale_body(f_ref, h0_ref, h1_ref, o_ref):
    cnt = h0_ref[0, 0, :] + h1_ref[0, 0, :]
    scale = lax.rsqrt(jnp.maximum(cnt, 1.0))
    o_ref[...] = f_ref[...] * scale[:, None]


def _out_body(a0_ref, a1_ref, h0_ref, h1_ref, w_ref, b_ref, o_ref):
    cnt = h0_ref[0, 0, :] + h1_ref[0, 0, :]
    inv = lax.rsqrt(jnp.maximum(cnt, 1.0))
    x = (a0_ref[...] + a1_ref[...]) * inv[:, None]
    y = jnp.dot(x, w_ref[...], preferred_element_type=jnp.float32)
    o_ref[...] = jnp.maximum(y + b_ref[0:1, :], 0.0)


def kernel(feature, edge_index, W, b):
    src = edge_index[0]
    dst = edge_index[1]
    pad = N + (jnp.arange(E_PAD - E, dtype=jnp.int32) % (NP - N))
    src2 = jnp.concatenate([src, pad]).reshape(E_PAD // K, K)
    dst2 = jnp.concatenate([dst, pad]).reshape(E_PAD // K, K)
    feature_p = jnp.pad(feature, ((0, NP - N), (0, 0)))
    zer_h = jnp.zeros((HB, 128), dtype=jnp.float32)
    zer_r = jnp.zeros((K, D), dtype=jnp.float32)
    iden = jnp.arange(HB, dtype=jnp.int32)
    b2 = jnp.broadcast_to(b, (8, D))

    hist_s, hist_d = _deg_kernel(src2, dst2, zer_h, iden)
    hist_s3 = hist_s.reshape(NC * GB, 1, 1024)
    hist_d3 = hist_d.reshape(NC * GB, 1, 1024)

    h = pl.pallas_call(
        _scale_body,
        grid=(GB,),
        in_specs=[pl.BlockSpec((1024, D), lambda i: (i, 0)),
                  pl.BlockSpec((1, 1, 1024), lambda i: (i, 0, 0)),
                  pl.BlockSpec((1, 1, 1024), lambda i: (i + GB, 0, 0))],
        out_specs=pl.BlockSpec((1024, D), lambda i: (i, 0)),
        out_shape=jax.ShapeDtypeStruct((NP, D), jnp.float32),
    )(feature_p, hist_s3, hist_s3)

    acc = _edge_kernel(src2, dst2, h, zer_r)

    out = pl.pallas_call(
        _out_body,
        grid=(GB,),
        in_specs=[pl.BlockSpec((1024, D), lambda i: (i, 0)),
                  pl.BlockSpec((1024, D), lambda i: (i + GB, 0)),
                  pl.BlockSpec((1, 1, 1024), lambda i: (i, 0, 0)),
                  pl.BlockSpec((1, 1, 1024), lambda i: (i + GB, 0, 0)),
                  pl.BlockSpec((128, D), lambda i: (0, 0)),
                  pl.BlockSpec((8, D), lambda i: (0, 0))],
        out_specs=pl.BlockSpec((1024, D), lambda i: (i, 0)),
        out_shape=jax.ShapeDtypeStruct((NP, D), jnp.float32),
    )(acc, acc, hist_d3, hist_d3, W, b2)

    return out[:N]
